# baseline (speedup 1.0000x reference)
import functools

import jax
import jax.numpy as jnp
from jax.experimental import pallas as pl
from jax.experimental.pallas import tpu as pltpu

K = 7
P = 3
LANES = 128


def _reduce_lanes(v, c, inv_c):
    half = c // 2
    s = v[:, :, :half] + v[:, :, half:]
    m = jnp.maximum(v[:, :, :half], v[:, :, half:])
    while half > LANES:
        half //= 2
        s = s[:, :, :half] + s[:, :, half:]
        m = jnp.maximum(m[:, :, :half], m[:, :, half:])
    return jnp.sum(s, axis=-1) * inv_c, jnp.max(m, axis=-1)


def _conv_from_pad(pad_b, s, h, w):
    t = jnp.dot(pad_b, s, preferred_element_type=jnp.float32)
    acc = t[0:h, 0:w]
    for dy in range(1, K):
        acc = acc + t[dy:dy + h, dy * w:dy * w + w]
    return jax.nn.sigmoid(acc)


def _build_band(w_ref, s_ref, w, wp):
    ci = jax.lax.broadcasted_iota(jnp.int32, (wp, w), 0)
    c = jax.lax.broadcasted_iota(jnp.int32, (wp, w), 1)
    dx = ci - c
    masks = [dx == k for k in range(K)]
    for m in range(2):
        for dy in range(K):
            t = jnp.zeros((wp, w), jnp.float32)
            for k in range(K):
                t = t + jnp.where(masks[k], w_ref[(m * K + dy) * K + k], 0.0)
            s_ref[m * wp:(m + 1) * wp, dy * w:(dy + 1) * w] = t


def _fused_body(w_ref, x_ref, o_ref, pad_ref, s_ref, *, nb, c, h, w, inv_c):
    wp = w + 2 * P

    @pl.when(pl.program_id(0) == 0)
    def _():
        _build_band(w_ref, s_ref, w, wp)

    pad_ref[...] = jnp.zeros_like(pad_ref)
    for b in range(nb):
        avg, mx = _reduce_lanes(x_ref[b], c, inv_c)
        pad_ref[b, P:P + h, P:P + w] = avg
        pad_ref[b, P:P + h, wp + P:wp + P + w] = mx
    s = s_ref[...]
    for b in range(nb):
        o_ref[b] = _conv_from_pad(pad_ref[b], s, h, w)


def _reduce_flat_body(x_ref, avg_ref, max_ref, *, inv_c):
    v = x_ref[...]
    avg_ref[0] = jnp.sum(v, axis=1) * inv_c
    max_ref[0] = jnp.max(v, axis=1)


def _conv_body(w_ref, avg_ref, max_ref, o_ref, pad_ref, s_ref, *, nb, h, w):
    wp = w + 2 * P

    @pl.when(pl.program_id(0) == 0)
    def _():
        _build_band(w_ref, s_ref, w, wp)

    pad_ref[...] = jnp.zeros_like(pad_ref)
    pad_ref[:, P:P + h, P:P + w] = avg_ref[...]
    pad_ref[:, P:P + h, wp + P:wp + P + w] = max_ref[...]
    s = s_ref[...]
    for b in range(nb):
        o_ref[b] = _conv_from_pad(pad_ref[b], s, h, w)


def _generic_path(x, w_flat):
    N, C, H, W = x.shape
    HW = H * W
    Hp, Wp = H + 2 * P, W + 2 * P
    nbc = 8 if N % 8 == 0 else 1
    avg, mx = pl.pallas_call(
        functools.partial(_reduce_flat_body, inv_c=1.0 / C),
        out_shape=(jax.ShapeDtypeStruct((N, 1, HW), jnp.float32),
                   jax.ShapeDtypeStruct((N, 1, HW), jnp.float32)),
        grid=(N,),
        in_specs=[pl.BlockSpec((1, C, HW), lambda i: (i, 0, 0))],
        out_specs=(pl.BlockSpec((1, 1, HW), lambda i: (i, 0, 0)),
                   pl.BlockSpec((1, 1, HW), lambda i: (i, 0, 0))),
        compiler_params=pltpu.CompilerParams(
            dimension_semantics=("parallel",),
            vmem_limit_bytes=48 << 20),
        cost_estimate=pl.CostEstimate(
            flops=2 * N * C * HW, transcendentals=0,
            bytes_accessed=(N * C * HW + 2 * N * HW) * 4),
    )(x.reshape(N, C, HW))
    return pl.pallas_call(
        functools.partial(_conv_body, nb=nbc, h=H, w=W),
        out_shape=jax.ShapeDtypeStruct((N, H, W), x.dtype),
        grid=(N // nbc,),
        in_specs=[
            pl.BlockSpec(memory_space=pltpu.MemorySpace.SMEM),
            pl.BlockSpec((nbc, H, W), lambda i: (i, 0, 0)),
            pl.BlockSpec((nbc, H, W), lambda i: (i, 0, 0)),
        ],
        out_specs=pl.BlockSpec((nbc, H, W), lambda i: (i, 0, 0)),
        scratch_shapes=[pltpu.VMEM((nbc, Hp, 2 * Wp), jnp.float32),
                        pltpu.VMEM((2 * Wp, K * W), jnp.float32)],
        compiler_params=pltpu.CompilerParams(
            dimension_semantics=("arbitrary",),
            vmem_limit_bytes=32 << 20),
        cost_estimate=pl.CostEstimate(
            flops=2 * N * Hp * 2 * Wp * K * W + 8 * N * HW,
            transcendentals=N * HW,
            bytes_accessed=(3 * N * HW) * 4),
    )(w_flat, avg.reshape(N, H, W), mx.reshape(N, H, W))


def kernel(x, conv_weight):
    N, C, H, W = x.shape
    HW = H * W
    Hp, Wp = H + 2 * P, W + 2 * P

    w_flat = conv_weight.reshape(2 * K * K).astype(jnp.float32)
    lanes_ok = C % LANES == 0 and ((C // LANES) & (C // LANES - 1)) == 0
    if not lanes_ok:
        return _generic_path(x, w_flat).reshape(N, 1, H, W)

    nb = 2 if N % 2 == 0 else 1
    xt = jnp.transpose(x, (0, 2, 3, 1))
    out = pl.pallas_call(
        functools.partial(_fused_body, nb=nb, c=C, h=H, w=W, inv_c=1.0 / C),
        out_shape=jax.ShapeDtypeStruct((N, H, W), x.dtype),
        grid=(N // nb,),
        in_specs=[
            pl.BlockSpec(memory_space=pltpu.MemorySpace.SMEM),
            pl.BlockSpec((nb, H, W, C), lambda i: (i, 0, 0, 0)),
        ],
        out_specs=pl.BlockSpec((nb, H, W), lambda i: (i, 0, 0)),
        scratch_shapes=[pltpu.VMEM((nb, Hp, 2 * Wp), jnp.float32),
                        pltpu.VMEM((2 * Wp, K * W), jnp.float32)],
        compiler_params=pltpu.CompilerParams(
            dimension_semantics=("arbitrary",),
            vmem_limit_bytes=48 << 20),
        cost_estimate=pl.CostEstimate(
            flops=2 * N * C * HW + 2 * N * Hp * 2 * Wp * K * W,
            transcendentals=N * HW,
            bytes_accessed=(N * C * HW + N * HW) * 4),
    )(w_flat, xt)
    return out.reshape(N, 1, H, W)


# speedup vs baseline: 1.0560x; 1.0560x over previous
import functools

import jax
import jax.numpy as jnp
from jax.experimental import pallas as pl
from jax.experimental.pallas import tpu as pltpu

K = 7
P = 3
LANES = 128


def _reduce_lanes(v, c, inv_c):
    half = c // 2
    s = v[:, :, :half] + v[:, :, half:]
    m = jnp.maximum(v[:, :, :half], v[:, :, half:])
    while half > LANES:
        half //= 2
        s = s[:, :, :half] + s[:, :, half:]
        m = jnp.maximum(m[:, :, :half], m[:, :, half:])
    return jnp.sum(s, axis=-1) * inv_c, jnp.max(m, axis=-1)


def _conv_from_pad(pad_b, s, h, w):
    t = jnp.dot(pad_b, s, preferred_element_type=jnp.float32)
    acc = t[0:h, 0:w]
    for dy in range(1, K):
        acc = acc + t[dy:dy + h, dy * w:dy * w + w]
    return jax.nn.sigmoid(acc)


def _build_band(w_ref, s_ref, w, wp):
    ci = jax.lax.broadcasted_iota(jnp.int32, (wp, w), 0)
    c = jax.lax.broadcasted_iota(jnp.int32, (wp, w), 1)
    dx = ci - c
    masks = [dx == k for k in range(K)]
    for m in range(2):
        for dy in range(K):
            t = jnp.zeros((wp, w), jnp.float32)
            for k in range(K):
                t = t + jnp.where(masks[k], w_ref[(m * K + dy) * K + k], 0.0)
            s_ref[m * wp:(m + 1) * wp, dy * w:(dy + 1) * w] = t


def _fused_body(w_ref, x_ref, o_ref, pad_ref, s_ref, *, nb, c, h, w, inv_c):
    wp = w + 2 * P

    @pl.when(pl.program_id(0) == 0)
    def _():
        _build_band(w_ref, s_ref, w, wp)

    pad_ref[...] = jnp.zeros_like(pad_ref)
    for b in range(nb):
        avg, mx = _reduce_lanes(x_ref[b], c, inv_c)
        pad_ref[b, P:P + h, P:P + w] = avg
        pad_ref[b, P:P + h, wp + P:wp + P + w] = mx
    s = s_ref[...]
    for b in range(nb):
        o_ref[b] = _conv_from_pad(pad_ref[b], s, h, w)


def _reduce_flat_body(x_ref, avg_ref, max_ref, *, inv_c):
    v = x_ref[...]
    avg_ref[0] = jnp.sum(v, axis=1) * inv_c
    max_ref[0] = jnp.max(v, axis=1)


def _conv_body(w_ref, avg_ref, max_ref, o_ref, pad_ref, s_ref, *, nb, h, w):
    wp = w + 2 * P

    @pl.when(pl.program_id(0) == 0)
    def _():
        _build_band(w_ref, s_ref, w, wp)

    pad_ref[...] = jnp.zeros_like(pad_ref)
    pad_ref[:, P:P + h, P:P + w] = avg_ref[...]
    pad_ref[:, P:P + h, wp + P:wp + P + w] = max_ref[...]
    s = s_ref[...]
    for b in range(nb):
        o_ref[b] = _conv_from_pad(pad_ref[b], s, h, w)


def _generic_path(x, w_flat):
    N, C, H, W = x.shape
    HW = H * W
    Hp, Wp = H + 2 * P, W + 2 * P
    nbc = 8 if N % 8 == 0 else 1
    avg, mx = pl.pallas_call(
        functools.partial(_reduce_flat_body, inv_c=1.0 / C),
        out_shape=(jax.ShapeDtypeStruct((N, 1, HW), jnp.float32),
                   jax.ShapeDtypeStruct((N, 1, HW), jnp.float32)),
        grid=(N,),
        in_specs=[pl.BlockSpec((1, C, HW), lambda i: (i, 0, 0))],
        out_specs=(pl.BlockSpec((1, 1, HW), lambda i: (i, 0, 0)),
                   pl.BlockSpec((1, 1, HW), lambda i: (i, 0, 0))),
        compiler_params=pltpu.CompilerParams(
            dimension_semantics=("parallel",),
            vmem_limit_bytes=60 << 20),
        cost_estimate=pl.CostEstimate(
            flops=2 * N * C * HW, transcendentals=0,
            bytes_accessed=(N * C * HW + 2 * N * HW) * 4),
    )(x.reshape(N, C, HW))
    return pl.pallas_call(
        functools.partial(_conv_body, nb=nbc, h=H, w=W),
        out_shape=jax.ShapeDtypeStruct((N, H, W), x.dtype),
        grid=(N // nbc,),
        in_specs=[
            pl.BlockSpec(memory_space=pltpu.MemorySpace.SMEM),
            pl.BlockSpec((nbc, H, W), lambda i: (i, 0, 0)),
            pl.BlockSpec((nbc, H, W), lambda i: (i, 0, 0)),
        ],
        out_specs=pl.BlockSpec((nbc, H, W), lambda i: (i, 0, 0)),
        scratch_shapes=[pltpu.VMEM((nbc, Hp, 2 * Wp), jnp.float32),
                        pltpu.VMEM((2 * Wp, K * W), jnp.float32)],
        compiler_params=pltpu.CompilerParams(
            dimension_semantics=("arbitrary",),
            vmem_limit_bytes=32 << 20),
        cost_estimate=pl.CostEstimate(
            flops=2 * N * Hp * 2 * Wp * K * W + 8 * N * HW,
            transcendentals=N * HW,
            bytes_accessed=(3 * N * HW) * 4),
    )(w_flat, avg.reshape(N, H, W), mx.reshape(N, H, W))


def kernel(x, conv_weight):
    N, C, H, W = x.shape
    HW = H * W
    Hp, Wp = H + 2 * P, W + 2 * P

    w_flat = conv_weight.reshape(2 * K * K).astype(jnp.float32)
    lanes_ok = C % LANES == 0 and ((C // LANES) & (C // LANES - 1)) == 0
    if not lanes_ok:
        return _generic_path(x, w_flat).reshape(N, 1, H, W)

    nb = 8 if N % 8 == 0 else 1
    xt = jnp.transpose(x, (0, 2, 3, 1))
    out = pl.pallas_call(
        functools.partial(_fused_body, nb=nb, c=C, h=H, w=W, inv_c=1.0 / C),
        out_shape=jax.ShapeDtypeStruct((N, H, W), x.dtype),
        grid=(N // nb,),
        in_specs=[
            pl.BlockSpec(memory_space=pltpu.MemorySpace.SMEM),
            pl.BlockSpec((nb, H, W, C), lambda i: (i, 0, 0, 0)),
        ],
        out_specs=pl.BlockSpec((nb, H, W), lambda i: (i, 0, 0)),
        scratch_shapes=[pltpu.VMEM((nb, Hp, 2 * Wp), jnp.float32),
                        pltpu.VMEM((2 * Wp, K * W), jnp.float32)],
        compiler_params=pltpu.CompilerParams(
            dimension_semantics=("arbitrary",),
            vmem_limit_bytes=60 << 20),
        cost_estimate=pl.CostEstimate(
            flops=2 * N * C * HW + 2 * N * Hp * 2 * Wp * K * W,
            transcendentals=N * HW,
            bytes_accessed=(N * C * HW + N * HW) * 4),
    )(w_flat, xt)
    return out.reshape(N, 1, H, W)


# speedup vs baseline: 1.1049x; 1.0463x over previous
import functools

import jax
import jax.numpy as jnp
from jax.experimental import pallas as pl
from jax.experimental.pallas import tpu as pltpu

K = 7
P = 3
LANES = 128


def _reduce_lanes(v, c, inv_c):
    half = c // 2
    s = v[:, :, :half] + v[:, :, half:]
    m = jnp.maximum(v[:, :, :half], v[:, :, half:])
    while half > LANES:
        half //= 2
        s = s[:, :, :half] + s[:, :, half:]
        m = jnp.maximum(m[:, :, :half], m[:, :, half:])
    return jnp.sum(s, axis=-1) * inv_c, jnp.max(m, axis=-1)


def _conv_from_pad(pad_b, s, h, w):
    t = jnp.dot(pad_b, s, preferred_element_type=jnp.float32)
    acc = t[0:h, 0:w]
    for dy in range(1, K):
        acc = acc + t[dy:dy + h, dy * w:dy * w + w]
    return jax.nn.sigmoid(acc)


def _build_band(w_ref, s_ref, w, wp):
    ci = jax.lax.broadcasted_iota(jnp.int32, (wp, w), 0)
    c = jax.lax.broadcasted_iota(jnp.int32, (wp, w), 1)
    dx = ci - c
    masks = [dx == k for k in range(K)]
    for m in range(2):
        for dy in range(K):
            t = jnp.zeros((wp, w), jnp.float32)
            for k in range(K):
                t = t + jnp.where(masks[k], w_ref[(m * K + dy) * K + k], 0.0)
            s_ref[m * wp:(m + 1) * wp, dy * w:(dy + 1) * w] = t


def _fused_body(w_ref, x_ref, o_ref, pad_ref, s_ref, *, nb, c, h, w, inv_c):
    wp = w + 2 * P

    @pl.when(pl.program_id(0) == 0)
    def _():
        _build_band(w_ref, s_ref, w, wp)

    pad_ref[...] = jnp.zeros_like(pad_ref)
    for b in range(nb):
        avg, mx = _reduce_lanes(x_ref[b], c, inv_c)
        pad_ref[b, P:P + h, P:P + w] = avg
        pad_ref[b, P:P + h, wp + P:wp + P + w] = mx
    s = s_ref[...]
    for b in range(nb):
        o_ref[b] = _conv_from_pad(pad_ref[b], s, h, w)


def _reduce_flat_body(x_ref, avg_ref, max_ref, *, inv_c):
    v = x_ref[...]
    avg_ref[0] = jnp.sum(v, axis=1) * inv_c
    max_ref[0] = jnp.max(v, axis=1)


def _conv_body(w_ref, avg_ref, max_ref, o_ref, pad_ref, s_ref, *, nb, h, w):
    wp = w + 2 * P

    @pl.when(pl.program_id(0) == 0)
    def _():
        _build_band(w_ref, s_ref, w, wp)

    pad_ref[...] = jnp.zeros_like(pad_ref)
    pad_ref[:, P:P + h, P:P + w] = avg_ref[...]
    pad_ref[:, P:P + h, wp + P:wp + P + w] = max_ref[...]
    s = s_ref[...]
    for b in range(nb):
        o_ref[b] = _conv_from_pad(pad_ref[b], s, h, w)


def _generic_path(x, w_flat):
    N, C, H, W = x.shape
    HW = H * W
    Hp, Wp = H + 2 * P, W + 2 * P
    nbc = 8 if N % 8 == 0 else 1
    avg, mx = pl.pallas_call(
        functools.partial(_reduce_flat_body, inv_c=1.0 / C),
        out_shape=(jax.ShapeDtypeStruct((N, 1, HW), jnp.float32),
                   jax.ShapeDtypeStruct((N, 1, HW), jnp.float32)),
        grid=(N,),
        in_specs=[pl.BlockSpec((1, C, HW), lambda i: (i, 0, 0))],
        out_specs=(pl.BlockSpec((1, 1, HW), lambda i: (i, 0, 0)),
                   pl.BlockSpec((1, 1, HW), lambda i: (i, 0, 0))),
        compiler_params=pltpu.CompilerParams(
            dimension_semantics=("parallel",),
            vmem_limit_bytes=48 << 20),
        cost_estimate=pl.CostEstimate(
            flops=2 * N * C * HW, transcendentals=0,
            bytes_accessed=(N * C * HW + 2 * N * HW) * 4),
    )(x.reshape(N, C, HW))
    return pl.pallas_call(
        functools.partial(_conv_body, nb=nbc, h=H, w=W),
        out_shape=jax.ShapeDtypeStruct((N, H, W), x.dtype),
        grid=(N // nbc,),
        in_specs=[
            pl.BlockSpec(memory_space=pltpu.MemorySpace.SMEM),
            pl.BlockSpec((nbc, H, W), lambda i: (i, 0, 0)),
            pl.BlockSpec((nbc, H, W), lambda i: (i, 0, 0)),
        ],
        out_specs=pl.BlockSpec((nbc, H, W), lambda i: (i, 0, 0)),
        scratch_shapes=[pltpu.VMEM((nbc, Hp, 2 * Wp), jnp.float32),
                        pltpu.VMEM((2 * Wp, K * W), jnp.float32)],
        compiler_params=pltpu.CompilerParams(
            dimension_semantics=("arbitrary",),
            vmem_limit_bytes=32 << 20),
        cost_estimate=pl.CostEstimate(
            flops=2 * N * Hp * 2 * Wp * K * W + 8 * N * HW,
            transcendentals=N * HW,
            bytes_accessed=(3 * N * HW) * 4),
    )(w_flat, avg.reshape(N, H, W), mx.reshape(N, H, W))


def kernel(x, conv_weight):
    N, C, H, W = x.shape
    HW = H * W
    Hp, Wp = H + 2 * P, W + 2 * P

    w_flat = conv_weight.reshape(2 * K * K).astype(jnp.float32)
    lanes_ok = C % LANES == 0 and ((C // LANES) & (C // LANES - 1)) == 0
    if not lanes_ok:
        return _generic_path(x, w_flat).reshape(N, 1, H, W)

    nb = 4 if N % 4 == 0 else 1
    xt = jnp.transpose(x, (0, 2, 3, 1))
    out = pl.pallas_call(
        functools.partial(_fused_body, nb=nb, c=C, h=H, w=W, inv_c=1.0 / C),
        out_shape=jax.ShapeDtypeStruct((N, H, W), x.dtype),
        grid=(N // nb,),
        in_specs=[
            pl.BlockSpec(memory_space=pltpu.MemorySpace.SMEM),
            pl.BlockSpec((nb, H, W, C), lambda i: (i, 0, 0, 0)),
        ],
        out_specs=pl.BlockSpec((nb, H, W), lambda i: (i, 0, 0)),
        scratch_shapes=[pltpu.VMEM((nb, Hp, 2 * Wp), jnp.float32),
                        pltpu.VMEM((2 * Wp, K * W), jnp.float32)],
        compiler_params=pltpu.CompilerParams(
            dimension_semantics=("arbitrary",),
            vmem_limit_bytes=48 << 20),
        cost_estimate=pl.CostEstimate(
            flops=2 * N * C * HW + 2 * N * Hp * 2 * Wp * K * W,
            transcendentals=N * HW,
            bytes_accessed=(N * C * HW + N * HW) * 4),
    )(w_flat, xt)
    return out.reshape(N, 1, H, W)
